# baseline (speedup 1.0000x reference)
import functools

import jax
import jax.numpy as jnp
from jax import lax
from jax.experimental import pallas as pl
from jax.experimental.pallas import tpu as pltpu
from jax.experimental.pallas import tpu_sc as plsc

LANES = 16
CHUNK = 100
NBUF = 3


@functools.lru_cache(maxsize=None)
def _build(batch, seq_len, dim):
  info = plsc.get_sparse_core_info()
  nc, ns = info.num_cores, info.num_subcores
  nw = nc * ns
  spw = batch // nw
  cps = seq_len // CHUNK

  mesh = plsc.VectorSubcoreMesh(core_axis_name="c", subcore_axis_name="s")

  @functools.partial(
      pl.kernel,
      mesh=mesh,
      out_type=jax.ShapeDtypeStruct((batch, seq_len, dim), jnp.float32),
      scratch_types=[
          pltpu.VMEM((spw * cps, CHUNK), jnp.int32),
          pltpu.VMEM((seq_len, dim), jnp.float32),
          pltpu.VMEM((NBUF, seq_len, dim), jnp.float32),
          pltpu.SemaphoreType.DMA((NBUF,)),
          pltpu.SemaphoreType.DMA((NBUF,)),
          pltpu.SemaphoreType.DMA((NBUF,)),
      ],
  )
  def emb(tokens_hbm, pos_hbm, x_hbm, out_hbm, idx_v, pos_v, rows_v,
          gsem0, gsem1, wsem):
    wid = lax.axis_index("s") * nc + lax.axis_index("c")
    base = wid * spw
    pltpu.sync_copy(x_hbm.at[pl.ds(base * cps, spw * cps)], idx_v)
    pltpu.sync_copy(pos_hbm.at[pl.ds(0, seq_len)], pos_v)

    gsems = (gsem0, gsem1)

    def start_gather(q, b):
      handles = []
      for h in range(cps):
        handles.append(pltpu.async_copy(
            tokens_hbm.at[idx_v.at[q * cps + h]],
            rows_v.at[b, pl.ds(h * CHUNK, CHUNK)],
            gsems[h].at[b]))
      return handles

    def start_wb(q, b):
      return pltpu.async_copy(rows_v.at[b], out_hbm.at[base + q], wsem.at[b])

    def add_half(b, h):
      def row_body(i, rcarry):
        for u in range(2):
          r = h * CHUNK + 2 * i + u
          for j in range(dim // LANES):
            sl = pl.ds(j * LANES, LANES)
            rows_v[b, r, sl] = rows_v[b, r, sl] + pos_v[r, sl]
        return rcarry

      lax.fori_loop(0, CHUNK // 2, row_body, 0)

    gh = {}
    wbh = {}
    for q in range(min(NBUF - 1, spw)):
      gh[q] = start_gather(q, q % NBUF)

    for q in range(spw):
      b = q % NBUF
      h0, h1 = gh.pop(q)
      h0.wait()
      add_half(b, 0)
      nxt = q + NBUF - 1
      if NBUF - 1 <= nxt < spw:
        if nxt - NBUF in wbh:
          wbh.pop(nxt - NBUF).wait()
        gh[nxt] = start_gather(nxt, nxt % NBUF)
      h1.wait()
      add_half(b, 1)
      wbh[q] = start_wb(q, b)

    for q in sorted(wbh):
      wbh[q].wait()

  return emb


def kernel(tokens, positions, x):
  b, s = x.shape
  _, dim = tokens.shape
  x2 = x.reshape(b * s // CHUNK, CHUNK)
  return _build(b, s, dim)(tokens, positions, x2)


# speedup vs baseline: 1.0059x; 1.0059x over previous
import functools

import jax
import jax.numpy as jnp
from jax import lax
from jax.experimental import pallas as pl
from jax.experimental.pallas import tpu as pltpu
from jax.experimental.pallas import tpu_sc as plsc

LANES = 16
CHUNK = 100
NBUF = 3


@functools.lru_cache(maxsize=None)
def _build(batch, seq_len, dim):
  info = plsc.get_sparse_core_info()
  nc, ns = info.num_cores, info.num_subcores
  nw = nc * ns
  spw = batch // nw
  cps = seq_len // CHUNK

  mesh = plsc.VectorSubcoreMesh(core_axis_name="c", subcore_axis_name="s")

  @functools.partial(
      pl.kernel,
      mesh=mesh,
      out_type=jax.ShapeDtypeStruct((batch, seq_len, dim), jnp.float32),
      scratch_types=[
          pltpu.VMEM((spw * cps, CHUNK), jnp.int32),
          pltpu.VMEM((seq_len, dim), jnp.float32),
          pltpu.VMEM((NBUF, seq_len, dim), jnp.float32),
          pltpu.SemaphoreType.DMA((NBUF,)),
          pltpu.SemaphoreType.DMA((NBUF,)),
          pltpu.SemaphoreType.DMA((NBUF,)),
          pltpu.SemaphoreType.DMA((2,)),
      ],
  )
  def emb(tokens_hbm, pos_hbm, x_hbm, out_hbm, idx_v, pos_v, rows_v,
          gsem0, gsem1, wsem, ssem):
    wid = lax.axis_index("s") * nc + lax.axis_index("c")
    base = wid * spw
    ih = pltpu.async_copy(
        x_hbm.at[pl.ds(base * cps, spw * cps)], idx_v, ssem.at[0])
    ph = pltpu.async_copy(pos_hbm.at[pl.ds(0, seq_len)], pos_v, ssem.at[1])
    ih.wait()

    gsems = (gsem0, gsem1)

    def start_gather(q, b):
      handles = []
      for h in range(cps):
        handles.append(pltpu.async_copy(
            tokens_hbm.at[idx_v.at[q * cps + h]],
            rows_v.at[b, pl.ds(h * CHUNK, CHUNK)],
            gsems[h].at[b]))
      return handles

    def start_wb(q, b):
      return pltpu.async_copy(rows_v.at[b], out_hbm.at[base + q], wsem.at[b])

    def add_half(b, h):
      def row_body(i, rcarry):
        for u in range(2):
          r = h * CHUNK + 2 * i + u
          for j in range(dim // LANES):
            sl = pl.ds(j * LANES, LANES)
            rows_v[b, r, sl] = rows_v[b, r, sl] + pos_v[r, sl]
        return rcarry

      lax.fori_loop(0, CHUNK // 2, row_body, 0)

    gh = {}
    wbh = {}
    for q in range(min(NBUF - 1, spw)):
      gh[q] = start_gather(q, q % NBUF)

    for q in range(spw):
      b = q % NBUF
      h0, h1 = gh.pop(q)
      h0.wait()
      if q == 0:
        ph.wait()
      add_half(b, 0)
      nxt = q + NBUF - 1
      if NBUF - 1 <= nxt < spw:
        if nxt - NBUF in wbh:
          wbh.pop(nxt - NBUF).wait()
        gh[nxt] = start_gather(nxt, nxt % NBUF)
      h1.wait()
      add_half(b, 1)
      wbh[q] = start_wb(q, b)

    for q in sorted(wbh):
      wbh[q].wait()

  return emb


def kernel(tokens, positions, x):
  b, s = x.shape
  _, dim = tokens.shape
  x2 = x.reshape(b * s // CHUNK, CHUNK)
  return _build(b, s, dim)(tokens, positions, x2)


# speedup vs baseline: 1.0801x; 1.0737x over previous
import functools

import jax
import jax.numpy as jnp
from jax import lax
from jax.experimental import pallas as pl
from jax.experimental.pallas import tpu as pltpu
from jax.experimental.pallas import tpu_sc as plsc

LANES = 16
CHUNK = 100
NBUF = 3


@functools.lru_cache(maxsize=None)
def _build(batch, seq_len, dim):
  info = plsc.get_sparse_core_info()
  nc, ns = info.num_cores, info.num_subcores
  nw = nc * ns
  spw = batch // nw
  cps = seq_len // CHUNK
  nblk = spw // NBUF
  tail = spw % NBUF

  mesh = plsc.VectorSubcoreMesh(core_axis_name="c", subcore_axis_name="s")

  @functools.partial(
      pl.kernel,
      mesh=mesh,
      out_type=(
          jax.ShapeDtypeStruct((batch, seq_len, dim), jnp.float32),
          jax.ShapeDtypeStruct((CHUNK, dim), jnp.float32),
      ),
      scratch_types=[
          pltpu.VMEM((spw * cps, CHUNK), jnp.int32),
          pltpu.VMEM((seq_len, dim), jnp.float32),
          pltpu.VMEM((NBUF, seq_len, dim), jnp.float32),
          pltpu.SemaphoreType.DMA((NBUF,)),
          pltpu.SemaphoreType.DMA((NBUF,)),
          pltpu.SemaphoreType.DMA((NBUF,)),
          pltpu.SemaphoreType.DMA((2,)),
      ],
  )
  def emb(tokens_hbm, pos_hbm, x_hbm, out_hbm, dummy_hbm, idx_v, pos_v,
          rows_v, gsem0, gsem1, wsem, ssem):
    wid = lax.axis_index("s") * nc + lax.axis_index("c")
    base = wid * spw
    ih = pltpu.async_copy(
        x_hbm.at[pl.ds(base * cps, spw * cps)], idx_v, ssem.at[0])
    ph = pltpu.async_copy(pos_hbm.at[pl.ds(0, seq_len)], pos_v, ssem.at[1])
    ih.wait()

    gsems = (gsem0, gsem1)

    def start_gather(q, b):
      for h in range(cps):
        pltpu.async_copy(
            tokens_hbm.at[idx_v.at[q * cps + h]],
            rows_v.at[b, pl.ds(h * CHUNK, CHUNK)],
            gsems[h].at[b])

    def wait_half(h, b):
      pltpu.make_async_copy(
          dummy_hbm, rows_v.at[b, pl.ds(h * CHUNK, CHUNK)],
          gsems[h].at[b]).wait()

    def start_wb(q, b):
      pltpu.async_copy(rows_v.at[b], out_hbm.at[base + q], wsem.at[b])

    def wait_wb(b):
      pltpu.make_async_copy(rows_v.at[b], out_hbm.at[0], wsem.at[b]).wait()

    def add_half(b, h):
      def row_body(i, rcarry):
        for u in range(2):
          r = h * CHUNK + 2 * i + u
          for j in range(dim // LANES):
            sl = pl.ds(j * LANES, LANES)
            rows_v[b, r, sl] = rows_v[b, r, sl] + pos_v[r, sl]
        return rcarry

      lax.fori_loop(0, CHUNK // 2, row_body, 0)

    def do_seq(q, b, prefetch):
      wait_half(0, b)
      add_half(b, 0)
      if prefetch is not None:
        nq, nb, first_use = prefetch
        if not first_use:
          wait_wb(nb)
        start_gather(nq, nb)
      wait_half(1, b)
      add_half(b, 1)
      start_wb(q, b)

    start_gather(0, 0)
    start_gather(1, 1)
    wait_half(0, 0)
    ph.wait()
    add_half(0, 0)
    start_gather(2, 2)
    wait_half(1, 0)
    add_half(0, 1)
    start_wb(0, 0)
    do_seq(1, 1, (3, 0, False))
    do_seq(2, 2, (4, 1, False))

    def block(t, carry):
      q0 = t * NBUF
      do_seq(q0 + 0, 0, (q0 + 2, 2, False))
      do_seq(q0 + 1, 1, (q0 + 3, 0, False))
      do_seq(q0 + 2, 2, (q0 + 4, 1, False))
      return carry

    lax.fori_loop(1, nblk, block, 0)

    for u in range(tail):
      q = nblk * NBUF + u
      do_seq(q, q % NBUF, None)
    for u in range(NBUF):
      wait_wb((spw - NBUF + u) % NBUF)

  return emb


def kernel(tokens, positions, x):
  b, s = x.shape
  _, dim = tokens.shape
  x2 = x.reshape(b * s // CHUNK, CHUNK)
  out, _ = _build(b, s, dim)(tokens, positions, x2)
  return out


# speedup vs baseline: 1.0910x; 1.0102x over previous
import functools

import jax
import jax.numpy as jnp
from jax import lax
from jax.experimental import pallas as pl
from jax.experimental.pallas import tpu as pltpu
from jax.experimental.pallas import tpu_sc as plsc

LANES = 16
CHUNK = 100
NBUF = 3


@functools.lru_cache(maxsize=None)
def _build(batch, seq_len, dim):
  info = plsc.get_sparse_core_info()
  nc, ns = info.num_cores, info.num_subcores
  nw = nc * ns
  spw = batch // nw
  cps = seq_len // CHUNK
  nblk = spw // NBUF
  tail = spw % NBUF

  mesh = plsc.VectorSubcoreMesh(core_axis_name="c", subcore_axis_name="s")

  @functools.partial(
      pl.kernel,
      mesh=mesh,
      out_type=(
          jax.ShapeDtypeStruct((batch, seq_len, dim), jnp.float32),
          jax.ShapeDtypeStruct((CHUNK, dim), jnp.float32),
      ),
      scratch_types=[
          pltpu.VMEM((spw * cps, CHUNK), jnp.int32),
          pltpu.VMEM((seq_len, dim), jnp.float32),
          pltpu.VMEM((NBUF, seq_len, dim), jnp.float32),
          pltpu.SemaphoreType.DMA((NBUF,)),
          pltpu.SemaphoreType.DMA((NBUF,)),
          pltpu.SemaphoreType.DMA((NBUF,)),
          pltpu.SemaphoreType.DMA((2,)),
      ],
  )
  def emb(tokens_hbm, pos_hbm, x_hbm, out_hbm, dummy_hbm, idx_v, pos_v,
          rows_v, gsem0, gsem1, wsem, ssem):
    wid = lax.axis_index("s") * nc + lax.axis_index("c")
    base = wid * spw
    ih = pltpu.async_copy(
        x_hbm.at[pl.ds(base * cps, spw * cps)], idx_v, ssem.at[0])
    ph = pltpu.async_copy(pos_hbm.at[pl.ds(0, seq_len)], pos_v, ssem.at[1])
    ih.wait()

    gsems = (gsem0, gsem1)

    def start_gather(q, b):
      for h in range(cps):
        pltpu.async_copy(
            tokens_hbm.at[idx_v.at[q * cps + h]],
            rows_v.at[b, pl.ds(h * CHUNK, CHUNK)],
            gsems[h].at[b])

    def wait_half(h, b):
      pltpu.make_async_copy(
          dummy_hbm, rows_v.at[b, pl.ds(h * CHUNK, CHUNK)],
          gsems[h].at[b]).wait()

    def start_wb(q, b):
      pltpu.async_copy(rows_v.at[b], out_hbm.at[base + q], wsem.at[b])

    def wait_wb(b):
      pltpu.make_async_copy(rows_v.at[b], out_hbm.at[0], wsem.at[b]).wait()

    def add_half(b, h):
      def row_body(i, rcarry):
        for u in range(2):
          r = h * CHUNK + 2 * i + u
          for j in range(dim // LANES):
            sl = pl.ds(j * LANES, LANES)
            rows_v[b, r, sl] = rows_v[b, r, sl] + pos_v[r, sl]
        return rcarry

      lax.fori_loop(0, CHUNK // 2, row_body, 0)

    def do_seq(q, b, prefetch):
      wait_half(0, b)
      add_half(b, 0)
      wait_half(1, b)
      add_half(b, 1)
      start_wb(q, b)
      if prefetch is not None:
        nq, nb, first_use = prefetch
        if not first_use:
          wait_wb(nb)
        start_gather(nq, nb)

    start_gather(0, 0)
    start_gather(1, 1)
    wait_half(0, 0)
    ph.wait()
    add_half(0, 0)
    start_gather(2, 2)
    wait_half(1, 0)
    add_half(0, 1)
    start_wb(0, 0)
    do_seq(1, 1, (3, 0, False))
    do_seq(2, 2, (4, 1, False))

    def block(t, carry):
      q0 = t * NBUF
      do_seq(q0 + 0, 0, (q0 + 2, 2, False))
      do_seq(q0 + 1, 1, (q0 + 3, 0, False))
      do_seq(q0 + 2, 2, (q0 + 4, 1, False))
      return carry

    lax.fori_loop(1, nblk, block, 0)

    for u in range(tail):
      q = nblk * NBUF + u
      do_seq(q, q % NBUF, None)
    for u in range(NBUF):
      wait_wb((spw - NBUF + u) % NBUF)

  return emb


def kernel(tokens, positions, x):
  b, s = x.shape
  _, dim = tokens.shape
  x2 = x.reshape(b * s // CHUNK, CHUNK)
  out, _ = _build(b, s, dim)(tokens, positions, x2)
  return out
